# Initial kernel scaffold; baseline (speedup 1.0000x reference)
import jax
import jax.numpy as jnp
from jax.experimental import pallas as pl


def kernel(query, edge_index, adj_values, Ww, bw, W1, b1, W2, b2, ln1_g, ln1_b, ln2_g, ln2_b):
    raise NotImplementedError("write your pallas kernel here")


# speedup vs baseline: 17.0370x; 17.0370x over previous
import functools

import jax
import jax.numpy as jnp
from jax import lax
from jax.experimental import pallas as pl
from jax.experimental.pallas import tpu as pltpu
from jax.experimental.pallas import tpu_sc as plsc

N = 10000
E = 320000
D = 128
DA = 144
NC, NS, L = 2, 16, 16
NW = NC * NS
EPT = E // NW
G = 80
NCH = EPT // G
NP_ = 10240
RPT = NP_ // NS
EPS = 1e-6


def _pre_body(x_ref, Ww_ref, bw_ref, W1_ref, b1_ref, W2_ref, b2_ref,
              g1_ref, be1_ref, va_ref, s1_ref, s2_ref):
    x = x_ref[...]
    mu = jnp.mean(x, axis=1, keepdims=True)
    var = jnp.mean(jnp.square(x - mu), axis=1, keepdims=True)
    q = (x - mu) * lax.rsqrt(var + EPS) * g1_ref[...] + be1_ref[...]
    value = jnp.dot(q, Ww_ref[...], preferred_element_type=jnp.float32) + bw_ref[...]
    at1 = jnp.dot(q, W1_ref[...], preferred_element_type=jnp.float32) + b1_ref[...]
    at2 = jnp.dot(q, W2_ref[...], preferred_element_type=jnp.float32) + b2_ref[...]
    s1_ref[...] = jnp.tanh(jnp.sum(at1, axis=1, keepdims=True))
    s2_ref[...] = jnp.tanh(jnp.sum(at2, axis=1, keepdims=True))
    va_ref[:, :D] = value
    lane = lax.broadcasted_iota(jnp.int32, (x.shape[0], DA - D), 1)
    va_ref[:, D:] = jnp.where(lane == 0, 1.0, 0.0)


_pre = pl.pallas_call(
    _pre_body,
    out_shape=[
        jax.ShapeDtypeStruct((N, DA), jnp.float32),
        jax.ShapeDtypeStruct((N, 1), jnp.float32),
        jax.ShapeDtypeStruct((N, 1), jnp.float32),
    ],
)


def _sc_body(pk_hbm, s1_hbm, s2_hbm, va_hbm, U_hbm,
             pk_v, s1_v, s2_v, w_v, rows_v, U_sh, sem):
    c = lax.axis_index("c")
    s = lax.axis_index("s")
    wid = c * NS + s

    pltpu.sync_copy(s1_hbm, s1_v)
    pltpu.sync_copy(s2_hbm, s2_v)

    def zrow(i, carry):
        for t in range(DA // L):
            rows_v[i, pl.ds(t * L, L)] = jnp.zeros((L,), jnp.float32)
        return carry
    lax.fori_loop(0, G, zrow, 0)
    for k in range(RPT // G):
        pltpu.sync_copy(rows_v, U_sh.at[pl.ds(s * RPT + k * G, G)])
    plsc.subcore_barrier()

    def chunk(j, carry):
        pltpu.sync_copy(pk_hbm.at[wid, j], pk_v)
        gat = pltpu.async_copy(va_hbm.at[pk_v.at[1]], rows_v, sem)
        for k in range(G // L):
            sl = pl.ds(k * L, L)
            r = pk_v[0, sl]
            cc = pk_v[1, sl]
            a = plsc.bitcast(pk_v[2, sl], jnp.float32)
            g1 = plsc.load_gather(s1_v, [r])
            g2 = plsc.load_gather(s2_v, [cc])
            x = a * g1 + a * g2
            x = jnp.where(x >= 0.0, x, 0.2 * x)
            w_v[sl] = jnp.exp(x)
        gat.wait()
        def edge(e, carry2):
            we = plsc.load_gather(w_v, [jnp.full((L,), e, jnp.int32)])
            for t in range(DA // L):
                sl2 = pl.ds(t * L, L)
                rows_v[e, sl2] = rows_v[e, sl2] * we
            return carry2
        lax.fori_loop(0, G, edge, 0)
        pltpu.sync_copy(rows_v, U_sh.at[pk_v.at[0]], add=True)
        return carry
    lax.fori_loop(0, NCH, chunk, 0)

    plsc.subcore_barrier()
    pltpu.sync_copy(U_sh.at[pl.ds(s * RPT, RPT)], U_hbm.at[c, pl.ds(s * RPT, RPT)])


_sc = pl.kernel(
    _sc_body,
    out_type=jax.ShapeDtypeStruct((NC, NP_, DA), jnp.float32),
    mesh=plsc.VectorSubcoreMesh(core_axis_name="c", subcore_axis_name="s"),
    scratch_types=[
        pltpu.VMEM((3, G), jnp.int32),
        pltpu.VMEM((N,), jnp.float32),
        pltpu.VMEM((N,), jnp.float32),
        pltpu.VMEM((G,), jnp.float32),
        pltpu.VMEM((G, DA), jnp.float32),
        pltpu.VMEM_SHARED((NP_, DA), jnp.float32),
        pltpu.SemaphoreType.DMA,
    ],
    compiler_params=pltpu.CompilerParams(needs_layout_passes=False,
                                         use_tc_tiling_on_sc=False),
)


def _post_body(U_ref, g2_ref, be2_ref, o_ref):
    Uall = U_ref[0, :N] + U_ref[1, :N]
    num = Uall[:, :D]
    den = Uall[:, D:D + 1]
    den = jnp.where(den == 0.0, 1.0, den)
    o = num / den
    mu = jnp.mean(o, axis=1, keepdims=True)
    var = jnp.mean(jnp.square(o - mu), axis=1, keepdims=True)
    o_ref[...] = (o - mu) * lax.rsqrt(var + EPS) * g2_ref[...] + be2_ref[...]


_post = pl.pallas_call(
    _post_body,
    out_shape=jax.ShapeDtypeStruct((N, D), jnp.float32),
)


def kernel(query, edge_index, adj_values, Ww, bw, W1, b1, W2, b2,
           ln1_g, ln1_b, ln2_g, ln2_b):
    row = edge_index[0].astype(jnp.int32)
    col = edge_index[1].astype(jnp.int32)
    adj_i = lax.bitcast_convert_type(adj_values.astype(jnp.float32), jnp.int32)
    pk = jnp.stack([row.reshape(NW, NCH, G), col.reshape(NW, NCH, G),
                    adj_i.reshape(NW, NCH, G)], axis=2)
    va, s1, s2 = _pre(query, Ww, bw.reshape(1, D), W1, b1.reshape(1, D),
                      W2, b2.reshape(1, D), ln1_g.reshape(1, D), ln1_b.reshape(1, D))
    U = _sc(pk, s1.reshape(N), s2.reshape(N), va)
    return _post(U, ln2_g.reshape(1, D), ln2_b.reshape(1, D))


# speedup vs baseline: 23.1411x; 1.3583x over previous
import functools

import jax
import jax.numpy as jnp
from jax import lax
from jax.experimental import pallas as pl
from jax.experimental.pallas import tpu as pltpu
from jax.experimental.pallas import tpu_sc as plsc

N = 10000
E = 320000
D = 128
DA = 144
NC, NS, L = 2, 16, 16
NW = NC * NS
EPT = E // NW
G = 80
NCH = EPT // G
NP_ = 10240
RPT = NP_ // NS
EPS = 1e-6


def _pre_body(x_ref, Ww_ref, bw_ref, W1_ref, b1_ref, W2_ref, b2_ref,
              g1_ref, be1_ref, va_ref, s1_ref):
    x = x_ref[...]
    mu = jnp.mean(x, axis=1, keepdims=True)
    var = jnp.mean(jnp.square(x - mu), axis=1, keepdims=True)
    q = (x - mu) * lax.rsqrt(var + EPS) * g1_ref[...] + be1_ref[...]
    value = jnp.dot(q, Ww_ref[...], preferred_element_type=jnp.float32) + bw_ref[...]
    at1 = jnp.dot(q, W1_ref[...], preferred_element_type=jnp.float32) + b1_ref[...]
    at2 = jnp.dot(q, W2_ref[...], preferred_element_type=jnp.float32) + b2_ref[...]
    s1_ref[...] = jnp.tanh(jnp.sum(at1, axis=1, keepdims=True))
    s2 = jnp.tanh(jnp.sum(at2, axis=1, keepdims=True))
    va_ref[:, :D] = value
    lane = lax.broadcasted_iota(jnp.int32, (x.shape[0], DA - D), 1)
    s2b = jnp.broadcast_to(s2, (x.shape[0], DA - D))
    va_ref[:, D:] = jnp.where(lane == 0, 1.0, jnp.where(lane == 1, s2b, 0.0))


_pre = pl.pallas_call(
    _pre_body,
    out_shape=[
        jax.ShapeDtypeStruct((N, DA), jnp.float32),
        jax.ShapeDtypeStruct((N, 1), jnp.float32),
    ],
)


def _sc_body(pk_hbm, s1_hbm, va_hbm, U_hbm,
             pk0, pk1, s1_v, w_v, rows0, rows1, U_sh, sem0, sem1):
    c = lax.axis_index("c")
    s = lax.axis_index("s")
    wid = c * NS + s

    pltpu.sync_copy(s1_hbm, s1_v)

    def zrow(i, carry):
        for t in range(DA // L):
            rows0[i, pl.ds(t * L, L)] = jnp.zeros((L,), jnp.float32)
        return carry
    lax.fori_loop(0, G, zrow, 0)
    for k in range(RPT // G):
        pltpu.sync_copy(rows0, U_sh.at[pl.ds(s * RPT + k * G, G)])
    plsc.subcore_barrier()

    def issue(j, pk_b, rows_b, sem_b):
        pltpu.sync_copy(pk_hbm.at[wid, j], pk_b)
        pltpu.async_copy(va_hbm.at[pk_b.at[1]], rows_b, sem_b)

    def process(pk_b, rows_b, sem_b):
        pltpu.make_async_copy(va_hbm.at[pk_b.at[1]], rows_b, sem_b).wait()
        c129 = jnp.full((L,), D + 1, jnp.int32)
        for k in range(G // L):
            sl = pl.ds(k * L, L)
            r = pk_b[0, sl]
            a = plsc.bitcast(pk_b[2, sl], jnp.float32)
            evec = lax.iota(jnp.int32, L) + (k * L)
            g1 = plsc.load_gather(s1_v, [r])
            g2 = plsc.load_gather(rows_b, [evec, c129])
            x = a * g1 + a * g2
            x = jnp.where(x >= 0.0, x, 0.2 * x)
            w_v[sl] = jnp.exp(x)
        def edge(e2, carry2):
            for u in range(2):
                e = e2 * 2 + u
                we = plsc.load_gather(w_v, [jnp.full((L,), e, jnp.int32)])
                for t in range(DA // L):
                    sl2 = pl.ds(t * L, L)
                    rows_b[e, sl2] = rows_b[e, sl2] * we
            return carry2
        lax.fori_loop(0, G // 2, edge, 0)
        pltpu.sync_copy(rows_b, U_sh.at[pk_b.at[0]], add=True)

    issue(0, pk0, rows0, sem0)
    issue(1, pk1, rows1, sem1)

    def pair(p, carry):
        j0 = 2 * p
        process(pk0, rows0, sem0)
        issue(j0 + 2, pk0, rows0, sem0)
        process(pk1, rows1, sem1)

        @pl.when(j0 + 3 < NCH)
        def _():
            issue(j0 + 3, pk1, rows1, sem1)
        return carry
    lax.fori_loop(0, (NCH - 1) // 2, pair, 0)
    process(pk0, rows0, sem0)

    plsc.subcore_barrier()
    pltpu.sync_copy(U_sh.at[pl.ds(s * RPT, RPT)], U_hbm.at[c, pl.ds(s * RPT, RPT)])


_sc = pl.kernel(
    _sc_body,
    out_type=jax.ShapeDtypeStruct((NC, NP_, DA), jnp.float32),
    mesh=plsc.VectorSubcoreMesh(core_axis_name="c", subcore_axis_name="s"),
    scratch_types=[
        pltpu.VMEM((3, G), jnp.int32),
        pltpu.VMEM((3, G), jnp.int32),
        pltpu.VMEM((N,), jnp.float32),
        pltpu.VMEM((G,), jnp.float32),
        pltpu.VMEM((G, DA), jnp.float32),
        pltpu.VMEM((G, DA), jnp.float32),
        pltpu.VMEM_SHARED((NP_, DA), jnp.float32),
        pltpu.SemaphoreType.DMA,
        pltpu.SemaphoreType.DMA,
    ],
    compiler_params=pltpu.CompilerParams(needs_layout_passes=False,
                                         use_tc_tiling_on_sc=False),
)


def _post_body(U_ref, g2_ref, be2_ref, o_ref):
    Uall = U_ref[0, :N] + U_ref[1, :N]
    num = Uall[:, :D]
    den = Uall[:, D:D + 1]
    den = jnp.where(den == 0.0, 1.0, den)
    o = num / den
    mu = jnp.mean(o, axis=1, keepdims=True)
    var = jnp.mean(jnp.square(o - mu), axis=1, keepdims=True)
    o_ref[...] = (o - mu) * lax.rsqrt(var + EPS) * g2_ref[...] + be2_ref[...]


_post = pl.pallas_call(
    _post_body,
    out_shape=jax.ShapeDtypeStruct((N, D), jnp.float32),
)


def kernel(query, edge_index, adj_values, Ww, bw, W1, b1, W2, b2,
           ln1_g, ln1_b, ln2_g, ln2_b):
    row = edge_index[0].astype(jnp.int32)
    col = edge_index[1].astype(jnp.int32)
    adj_i = lax.bitcast_convert_type(adj_values.astype(jnp.float32), jnp.int32)
    pk = jnp.stack([row.reshape(NW, NCH, G), col.reshape(NW, NCH, G),
                    adj_i.reshape(NW, NCH, G)], axis=2)
    va, s1 = _pre(query, Ww, bw.reshape(1, D), W1, b1.reshape(1, D),
                  W2, b2.reshape(1, D), ln1_g.reshape(1, D), ln1_b.reshape(1, D))
    U = _sc(pk, s1.reshape(N), va)
    return _post(U, ln2_g.reshape(1, D), ln2_b.reshape(1, D))


# speedup vs baseline: 24.2859x; 1.0495x over previous
import functools

import jax
import jax.numpy as jnp
from jax import lax
from jax.experimental import pallas as pl
from jax.experimental.pallas import tpu as pltpu
from jax.experimental.pallas import tpu_sc as plsc

N = 10000
E = 320000
D = 128
DA = 144
NC, NS, L = 2, 16, 16
NW = NC * NS
EPT = E // NW
G = 80
NCH = EPT // G
NP_ = 10240
RPT = NP_ // NS
EPS = 1e-6


def _pre_body(x_ref, Ww_ref, bw_ref, W1_ref, b1_ref, W2_ref, b2_ref,
              g1_ref, be1_ref, va_ref, s1_ref):
    x = x_ref[...]
    mu = jnp.mean(x, axis=1, keepdims=True)
    var = jnp.mean(jnp.square(x - mu), axis=1, keepdims=True)
    q = (x - mu) * lax.rsqrt(var + EPS) * g1_ref[...] + be1_ref[...]
    value = jnp.dot(q, Ww_ref[...], preferred_element_type=jnp.float32) + bw_ref[...]
    at1 = jnp.dot(q, W1_ref[...], preferred_element_type=jnp.float32) + b1_ref[...]
    at2 = jnp.dot(q, W2_ref[...], preferred_element_type=jnp.float32) + b2_ref[...]
    s1_ref[...] = jnp.tanh(jnp.sum(at1, axis=1, keepdims=True))
    s2 = jnp.tanh(jnp.sum(at2, axis=1, keepdims=True))
    va_ref[:, :D] = value
    lane = lax.broadcasted_iota(jnp.int32, (x.shape[0], DA - D), 1)
    s2b = jnp.broadcast_to(s2, (x.shape[0], DA - D))
    va_ref[:, D:] = jnp.where(lane == 0, 1.0, jnp.where(lane == 1, s2b, 0.0))


_pre = pl.pallas_call(
    _pre_body,
    out_shape=[
        jax.ShapeDtypeStruct((N, DA), jnp.float32),
        jax.ShapeDtypeStruct((N, 1), jnp.float32),
    ],
)


def _sc_body(pk_hbm, s1_hbm, va_hbm, U_hbm,
             pk0, pk1, s1_v, w_v, rows0, rows1, U_sh, sem0, sem1, sem0s, sem1s):
    c = lax.axis_index("c")
    s = lax.axis_index("s")
    wid = c * NS + s

    pltpu.sync_copy(s1_hbm, s1_v)

    def zrow(i, carry):
        for t in range(DA // L):
            rows0[i, pl.ds(t * L, L)] = jnp.zeros((L,), jnp.float32)
        return carry
    lax.fori_loop(0, G, zrow, 0)
    for k in range(RPT // G):
        pltpu.sync_copy(rows0, U_sh.at[pl.ds(s * RPT + k * G, G)])
    plsc.subcore_barrier()

    def issue(j, pk_b, rows_b, sem_b):
        pltpu.sync_copy(pk_hbm.at[wid, j], pk_b)
        pltpu.async_copy(va_hbm.at[pk_b.at[1]], rows_b, sem_b)

    def process(pk_b, rows_b, sem_b, sem_s):
        pltpu.make_async_copy(va_hbm.at[pk_b.at[1]], rows_b, sem_b).wait()
        c129 = jnp.full((L,), D + 1, jnp.int32)
        for k in range(G // L):
            sl = pl.ds(k * L, L)
            r = pk_b[0, sl]
            a = plsc.bitcast(pk_b[2, sl], jnp.float32)
            evec = lax.iota(jnp.int32, L) + (k * L)
            g1 = plsc.load_gather(s1_v, [r])
            g2 = plsc.load_gather(rows_b, [evec, c129])
            x = a * g1 + a * g2
            x = jnp.where(x >= 0.0, x, 0.2 * x)
            w_v[sl] = jnp.exp(x)
        def edge(e2, carry2):
            for u in range(2):
                e = e2 * 2 + u
                we = plsc.load_gather(w_v, [jnp.full((L,), e, jnp.int32)])
                for t in range(DA // L):
                    sl2 = pl.ds(t * L, L)
                    rows_b[e, sl2] = rows_b[e, sl2] * we
            return carry2
        lax.fori_loop(0, G // 2, edge, 0)
        pltpu.async_copy(rows_b, U_sh.at[pk_b.at[0]], sem_s, add=True)

    def scatter_wait(pk_b, rows_b, sem_s):
        pltpu.make_async_copy(rows_b, U_sh.at[pk_b.at[0]], sem_s).wait()

    issue(0, pk0, rows0, sem0)
    issue(1, pk1, rows1, sem1)

    def pair(p, carry):
        j0 = 2 * p
        process(pk0, rows0, sem0, sem0s)
        process(pk1, rows1, sem1, sem1s)
        scatter_wait(pk0, rows0, sem0s)
        issue(j0 + 2, pk0, rows0, sem0)
        scatter_wait(pk1, rows1, sem1s)

        @pl.when(j0 + 3 < NCH)
        def _():
            issue(j0 + 3, pk1, rows1, sem1)
        return carry
    lax.fori_loop(0, (NCH - 1) // 2, pair, 0)
    process(pk0, rows0, sem0, sem0s)
    scatter_wait(pk0, rows0, sem0s)

    plsc.subcore_barrier()
    pltpu.sync_copy(U_sh.at[pl.ds(s * RPT, RPT)], U_hbm.at[c, pl.ds(s * RPT, RPT)])


_sc = pl.kernel(
    _sc_body,
    out_type=jax.ShapeDtypeStruct((NC, NP_, DA), jnp.float32),
    mesh=plsc.VectorSubcoreMesh(core_axis_name="c", subcore_axis_name="s"),
    scratch_types=[
        pltpu.VMEM((3, G), jnp.int32),
        pltpu.VMEM((3, G), jnp.int32),
        pltpu.VMEM((N,), jnp.float32),
        pltpu.VMEM((G,), jnp.float32),
        pltpu.VMEM((G, DA), jnp.float32),
        pltpu.VMEM((G, DA), jnp.float32),
        pltpu.VMEM_SHARED((NP_, DA), jnp.float32),
        pltpu.SemaphoreType.DMA,
        pltpu.SemaphoreType.DMA,
        pltpu.SemaphoreType.DMA,
        pltpu.SemaphoreType.DMA,
    ],
    compiler_params=pltpu.CompilerParams(needs_layout_passes=False,
                                         use_tc_tiling_on_sc=False),
)


def _post_body(U_ref, g2_ref, be2_ref, o_ref):
    Uall = U_ref[0, :N] + U_ref[1, :N]
    num = Uall[:, :D]
    den = Uall[:, D:D + 1]
    den = jnp.where(den == 0.0, 1.0, den)
    o = num / den
    mu = jnp.mean(o, axis=1, keepdims=True)
    var = jnp.mean(jnp.square(o - mu), axis=1, keepdims=True)
    o_ref[...] = (o - mu) * lax.rsqrt(var + EPS) * g2_ref[...] + be2_ref[...]


_post = pl.pallas_call(
    _post_body,
    out_shape=jax.ShapeDtypeStruct((N, D), jnp.float32),
)


def kernel(query, edge_index, adj_values, Ww, bw, W1, b1, W2, b2,
           ln1_g, ln1_b, ln2_g, ln2_b):
    row = edge_index[0].astype(jnp.int32)
    col = edge_index[1].astype(jnp.int32)
    adj_i = lax.bitcast_convert_type(adj_values.astype(jnp.float32), jnp.int32)
    pk = jnp.stack([row.reshape(NW, NCH, G), col.reshape(NW, NCH, G),
                    adj_i.reshape(NW, NCH, G)], axis=2)
    va, s1 = _pre(query, Ww, bw.reshape(1, D), W1, b1.reshape(1, D),
                  W2, b2.reshape(1, D), ln1_g.reshape(1, D), ln1_b.reshape(1, D))
    U = _sc(pk, s1.reshape(N), va)
    return _post(U, ln2_g.reshape(1, D), ln2_b.reshape(1, D))


# speedup vs baseline: 31.2433x; 1.2865x over previous
import functools

import jax
import jax.numpy as jnp
from jax import lax
from jax.experimental import pallas as pl
from jax.experimental.pallas import tpu as pltpu
from jax.experimental.pallas import tpu_sc as plsc

N = 10000
E = 320000
D = 128
DA = 144
NC, NS, L = 2, 16, 16
NW = NC * NS
EPT = E // NW
G = 80
NCH = EPT // G
NP_ = 10240
RPT = NP_ // NS
EPS = 1e-6


def _pre_body(x_ref, Ww_ref, bw_ref, W1_ref, b1_ref, W2_ref, b2_ref,
              g1_ref, be1_ref, va_ref, s1_ref):
    x = x_ref[...]
    mu = jnp.mean(x, axis=1, keepdims=True)
    var = jnp.mean(jnp.square(x - mu), axis=1, keepdims=True)
    q = (x - mu) * lax.rsqrt(var + EPS) * g1_ref[...] + be1_ref[...]
    value = jnp.dot(q, Ww_ref[...], preferred_element_type=jnp.float32) + bw_ref[...]
    at1 = jnp.dot(q, W1_ref[...], preferred_element_type=jnp.float32) + b1_ref[...]
    at2 = jnp.dot(q, W2_ref[...], preferred_element_type=jnp.float32) + b2_ref[...]
    s1_ref[...] = jnp.tanh(jnp.sum(at1, axis=1, keepdims=True))
    s2 = jnp.tanh(jnp.sum(at2, axis=1, keepdims=True))
    va_ref[:, :D] = value
    lane = lax.broadcasted_iota(jnp.int32, (x.shape[0], DA - D), 1)
    s2b = jnp.broadcast_to(s2, (x.shape[0], DA - D))
    va_ref[:, D:] = jnp.where(lane == 0, 1.0, jnp.where(lane == 1, s2b, 0.0))


_pre = pl.pallas_call(
    _pre_body,
    out_shape=[
        jax.ShapeDtypeStruct((N, DA), jnp.float32),
        jax.ShapeDtypeStruct((N, 1), jnp.float32),
    ],
)


def _sc_body(ei_hbm, adj_hbm, s1_hbm, va_hbm, U_hbm,
             r0, r1, c0, c1, a0, a1, s1_v, w_v, rows0, rows1, U_sh,
             semG0, semG1, semS0, semS1, semR0, semR1, semP0, semP1):
    c = lax.axis_index("c")
    s = lax.axis_index("s")
    wid = c * NS + s
    buf = ((r0, c0, a0, rows0, semG0, semS0, semR0, semP0),
           (r1, c1, a1, rows1, semG1, semS1, semR1, semP1))

    pltpu.async_copy(s1_hbm, s1_v, semG0)

    def zrow(i, carry):
        for t in range(DA // L):
            rows0[i, pl.ds(t * L, L)] = jnp.zeros((L,), jnp.float32)
        return carry
    lax.fori_loop(0, G, zrow, 0)
    for k in range(RPT // G):
        pltpu.async_copy(rows0, U_sh.at[pl.ds(s * RPT + k * G, G)], semS0)
    for k in range(RPT // G):
        pltpu.make_async_copy(rows0, U_sh.at[pl.ds(s * RPT + k * G, G)],
                              semS0).wait()
    pltpu.make_async_copy(s1_hbm, s1_v, semG0).wait()
    plsc.subcore_barrier()

    def fire_ca(j, b):
        _, c_b, a_b, _, _, _, _, semP = buf[b]
        pltpu.async_copy(ei_hbm.at[1, wid, j], c_b, semP)
        pltpu.async_copy(adj_hbm.at[wid, j], a_b, semP)

    def fire_r(j, b):
        r_b, _, _, _, _, _, semR, _ = buf[b]
        pltpu.async_copy(ei_hbm.at[0, wid, j], r_b, semR)

    def fire_gather(j, b):
        _, c_b, a_b, rows_b, semG, _, _, semP = buf[b]
        pltpu.make_async_copy(ei_hbm.at[1, wid, j], c_b, semP).wait()
        pltpu.make_async_copy(adj_hbm.at[wid, j], a_b, semP).wait()
        pltpu.async_copy(va_hbm.at[c_b], rows_b, semG)

    def process(j, b, prefetch, guard):
        r_b, c_b, a_b, rows_b, semG, semS, semR, _ = buf[b]
        pltpu.make_async_copy(ei_hbm.at[0, wid, j], r_b, semR).wait()
        pltpu.make_async_copy(va_hbm.at[c_b], rows_b, semG).wait()
        c129 = jnp.full((L,), D + 1, jnp.int32)
        for k in range(G // L):
            sl = pl.ds(k * L, L)
            r = r_b[sl]
            a = a_b[sl]
            evec = lax.iota(jnp.int32, L) + (k * L)
            g1 = plsc.load_gather(s1_v, [r])
            g2 = plsc.load_gather(rows_b, [evec, c129])
            x = a * g1 + a * g2
            x = jnp.where(x >= 0.0, x, 0.2 * x)
            w_v[sl] = jnp.exp(x)
        if prefetch:
            @pl.when(guard)
            def _():
                fire_ca(j + 2, b)
        def edge(e2, carry2):
            for u in range(2):
                e = e2 * 2 + u
                we = plsc.load_gather(w_v, [jnp.full((L,), e, jnp.int32)])
                for t in range(DA // L):
                    sl2 = pl.ds(t * L, L)
                    rows_b[e, sl2] = rows_b[e, sl2] * we
            return carry2
        lax.fori_loop(0, G // 2, edge, 0)
        pltpu.async_copy(rows_b, U_sh.at[r_b], semS, add=True)

    def scatter_wait(b):
        r_b, _, _, rows_b, _, semS, _, _ = buf[b]
        pltpu.make_async_copy(rows_b, U_sh.at[r_b], semS).wait()

    fire_r(0, 0)
    fire_ca(0, 0)
    fire_r(1, 1)
    fire_ca(1, 1)
    fire_gather(0, 0)
    fire_gather(1, 1)

    def pair(p, carry):
        j0 = 2 * p
        process(j0, 0, True, j0 + 2 < NCH)
        process(j0 + 1, 1, True, j0 + 3 < NCH)
        scatter_wait(0)
        fire_r(j0 + 2, 0)
        fire_gather(j0 + 2, 0)
        scatter_wait(1)

        @pl.when(j0 + 3 < NCH)
        def _():
            fire_r(j0 + 3, 1)
            fire_gather(j0 + 3, 1)
        return carry
    lax.fori_loop(0, (NCH - 1) // 2, pair, 0)
    process(NCH - 1, 0, False, True)
    scatter_wait(0)

    plsc.subcore_barrier()
    pltpu.sync_copy(U_sh.at[pl.ds(s * RPT, RPT)], U_hbm.at[c, pl.ds(s * RPT, RPT)])


_sc = pl.kernel(
    _sc_body,
    out_type=jax.ShapeDtypeStruct((NC, NP_, DA), jnp.float32),
    mesh=plsc.VectorSubcoreMesh(core_axis_name="c", subcore_axis_name="s"),
    scratch_types=[
        pltpu.VMEM((G,), jnp.int32),
        pltpu.VMEM((G,), jnp.int32),
        pltpu.VMEM((G,), jnp.int32),
        pltpu.VMEM((G,), jnp.int32),
        pltpu.VMEM((G,), jnp.float32),
        pltpu.VMEM((G,), jnp.float32),
        pltpu.VMEM((N,), jnp.float32),
        pltpu.VMEM((G,), jnp.float32),
        pltpu.VMEM((G, DA), jnp.float32),
        pltpu.VMEM((G, DA), jnp.float32),
        pltpu.VMEM_SHARED((NP_, DA), jnp.float32),
        pltpu.SemaphoreType.DMA,
        pltpu.SemaphoreType.DMA,
        pltpu.SemaphoreType.DMA,
        pltpu.SemaphoreType.DMA,
        pltpu.SemaphoreType.DMA,
        pltpu.SemaphoreType.DMA,
        pltpu.SemaphoreType.DMA,
        pltpu.SemaphoreType.DMA,
    ],
    compiler_params=pltpu.CompilerParams(needs_layout_passes=False,
                                         use_tc_tiling_on_sc=False),
)


def _post_body(U_ref, g2_ref, be2_ref, o_ref):
    Uall = U_ref[0, :N] + U_ref[1, :N]
    num = Uall[:, :D]
    den = Uall[:, D:D + 1]
    den = jnp.where(den == 0.0, 1.0, den)
    o = num / den
    mu = jnp.mean(o, axis=1, keepdims=True)
    var = jnp.mean(jnp.square(o - mu), axis=1, keepdims=True)
    o_ref[...] = (o - mu) * lax.rsqrt(var + EPS) * g2_ref[...] + be2_ref[...]


_post = pl.pallas_call(
    _post_body,
    out_shape=jax.ShapeDtypeStruct((N, D), jnp.float32),
)


def kernel(query, edge_index, adj_values, Ww, bw, W1, b1, W2, b2,
           ln1_g, ln1_b, ln2_g, ln2_b):
    ei = edge_index.astype(jnp.int32).reshape(2, NW, NCH, G)
    adjr = adj_values.astype(jnp.float32).reshape(NW, NCH, G)
    va, s1 = _pre(query, Ww, bw.reshape(1, D), W1, b1.reshape(1, D),
                  W2, b2.reshape(1, D), ln1_g.reshape(1, D), ln1_b.reshape(1, D))
    U = _sc(ei, adjr, s1.reshape(N), va)
    return _post(U, ln2_g.reshape(1, D), ln2_b.reshape(1, D))


# speedup vs baseline: 32.7738x; 1.0490x over previous
import functools

import jax
import jax.numpy as jnp
from jax import lax
from jax.experimental import pallas as pl
from jax.experimental.pallas import tpu as pltpu
from jax.experimental.pallas import tpu_sc as plsc

N = 10000
E = 320000
D = 128
DA = 144
NC, NS, L = 2, 16, 16
NW = NC * NS
EPT = E // NW
G = 80
NCH = EPT // G
NP_ = 10240
RPT = NP_ // NS
EPS = 1e-6


def _pre_body(x_ref, Ww_ref, bw_ref, W1_ref, b1_ref, W2_ref, b2_ref,
              g1_ref, be1_ref, va_ref, s1_ref):
    x = x_ref[...]
    mu = jnp.mean(x, axis=1, keepdims=True)
    var = jnp.mean(jnp.square(x - mu), axis=1, keepdims=True)
    q = (x - mu) * lax.rsqrt(var + EPS) * g1_ref[...] + be1_ref[...]
    value = jnp.dot(q, Ww_ref[...], preferred_element_type=jnp.float32) + bw_ref[...]
    at1 = jnp.dot(q, W1_ref[...], preferred_element_type=jnp.float32) + b1_ref[...]
    at2 = jnp.dot(q, W2_ref[...], preferred_element_type=jnp.float32) + b2_ref[...]
    s1_ref[...] = jnp.tanh(jnp.sum(at1, axis=1, keepdims=True))
    s2 = jnp.tanh(jnp.sum(at2, axis=1, keepdims=True))
    va_ref[:, :D] = value
    lane = lax.broadcasted_iota(jnp.int32, (x.shape[0], DA - D), 1)
    s2b = jnp.broadcast_to(s2, (x.shape[0], DA - D))
    va_ref[:, D:] = jnp.where(lane == 0, 1.0, jnp.where(lane == 1, s2b, 0.0))


_pre = pl.pallas_call(
    _pre_body,
    out_shape=[
        jax.ShapeDtypeStruct((N, DA), jnp.float32),
        jax.ShapeDtypeStruct((N, 1), jnp.float32),
    ],
)


def _sc_body(ei_hbm, adj_hbm, s1_hbm, va_hbm, U_hbm,
             r0, r1, r2, c0, c1, c2, a0, a1, a2, s1g0, s1g1, s1g2,
             w_v, rows0, rows1, rows2, U_sh,
             semG0, semG1, semG2, semS0, semS1, semS2,
             semR0, semR1, semR2, semP0, semP1, semP2):
    c = lax.axis_index("c")
    s = lax.axis_index("s")
    wid = c * NS + s
    buf = ((r0, c0, a0, s1g0, rows0, semG0, semS0, semR0, semP0),
           (r1, c1, a1, s1g1, rows1, semG1, semS1, semR1, semP1),
           (r2, c2, a2, s1g2, rows2, semG2, semS2, semR2, semP2))

    def zrow(i, carry):
        for t in range(DA // L):
            rows0[i, pl.ds(t * L, L)] = jnp.zeros((L,), jnp.float32)
        return carry
    lax.fori_loop(0, G, zrow, 0)
    for k in range(RPT // G):
        pltpu.async_copy(rows0, U_sh.at[pl.ds(s * RPT + k * G, G)], semS0)
    for k in range(RPT // G):
        pltpu.make_async_copy(rows0, U_sh.at[pl.ds(s * RPT + k * G, G)],
                              semS0).wait()
    plsc.subcore_barrier()

    def fire_rca(j, b):
        r_b, c_b, a_b, _, _, _, _, semR, semP = buf[b]
        pltpu.async_copy(ei_hbm.at[0, wid, j], r_b, semR)
        pltpu.async_copy(ei_hbm.at[1, wid, j], c_b, semP)
        pltpu.async_copy(adj_hbm.at[wid, j], a_b, semP)

    def fire_gathers(j, b):
        r_b, c_b, a_b, s1g_b, rows_b, semG, _, semR, semP = buf[b]
        pltpu.make_async_copy(ei_hbm.at[0, wid, j], r_b, semR).wait()
        pltpu.make_async_copy(ei_hbm.at[1, wid, j], c_b, semP).wait()
        pltpu.make_async_copy(adj_hbm.at[wid, j], a_b, semP).wait()
        pltpu.async_copy(va_hbm.at[c_b], rows_b, semG)
        pltpu.async_copy(s1_hbm.at[r_b], s1g_b, semG)

    def process(j, b, pre_j, scatter_guard):
        r_b, c_b, a_b, s1g_b, rows_b, semG, semS, _, _ = buf[b]
        b2 = (b + 2) % 3
        pltpu.make_async_copy(va_hbm.at[c_b], rows_b, semG).wait()
        pltpu.make_async_copy(s1_hbm.at[r_b], s1g_b, semG).wait()
        if scatter_guard is not None:
            @pl.when(scatter_guard)
            def _():
                scatter_wait(b2)
        else:
            scatter_wait(b2)
        if pre_j is not None:
            fire_rca(pre_j, b2)
        c129 = jnp.full((L,), D + 1, jnp.int32)
        for k in range(G // L):
            sl = pl.ds(k * L, L)
            a = a_b[sl]
            evec = lax.iota(jnp.int32, L) + (k * L)
            g1 = s1g_b[sl]
            g2 = plsc.load_gather(rows_b, [evec, c129])
            x = a * g1 + a * g2
            x = jnp.where(x >= 0.0, x, 0.2 * x)
            w_v[sl] = jnp.exp(x)
        def edge(e2, carry2):
            for u in range(2):
                e = e2 * 2 + u
                we = plsc.load_gather(w_v, [jnp.full((L,), e, jnp.int32)])
                for t in range(DA // L):
                    sl2 = pl.ds(t * L, L)
                    rows_b[e, sl2] = rows_b[e, sl2] * we
            return carry2
        lax.fori_loop(0, G // 2, edge, 0)
        if pre_j is not None:
            fire_gathers(pre_j, b2)
        pltpu.async_copy(rows_b, U_sh.at[r_b], semS, add=True)

    def scatter_wait(b):
        r_b, _, _, _, rows_b, _, semS, _, _ = buf[b]
        pltpu.make_async_copy(rows_b, U_sh.at[r_b], semS).wait()

    fire_rca(0, 0)
    fire_rca(1, 1)
    fire_gathers(0, 0)
    fire_gathers(1, 1)

    def tri(t, carry):
        j0 = 3 * t
        process(j0, 0, j0 + 2, t > 0)
        process(j0 + 1, 1, j0 + 3, None)
        process(j0 + 2, 2, j0 + 4, None)
        return carry
    lax.fori_loop(0, (NCH - 2) // 3, tri, 0)
    process(NCH - 2, 0, None, None)
    process(NCH - 1, 1, None, None)
    scatter_wait(1)

    plsc.subcore_barrier()
    pltpu.sync_copy(U_sh.at[pl.ds(s * RPT, RPT)], U_hbm.at[c, pl.ds(s * RPT, RPT)])


_sc = pl.kernel(
    _sc_body,
    out_type=jax.ShapeDtypeStruct((NC, NP_, DA), jnp.float32),
    mesh=plsc.VectorSubcoreMesh(core_axis_name="c", subcore_axis_name="s"),
    scratch_types=(
        [pltpu.VMEM((G,), jnp.int32)] * 6
        + [pltpu.VMEM((G,), jnp.float32)] * 6
        + [pltpu.VMEM((G,), jnp.float32)]
        + [pltpu.VMEM((G, DA), jnp.float32)] * 3
        + [pltpu.VMEM_SHARED((NP_, DA), jnp.float32)]
        + [pltpu.SemaphoreType.DMA] * 12
    ),
    compiler_params=pltpu.CompilerParams(needs_layout_passes=False,
                                         use_tc_tiling_on_sc=False),
)


def _post_body(U_ref, g2_ref, be2_ref, o_ref):
    Uall = U_ref[0, :N] + U_ref[1, :N]
    num = Uall[:, :D]
    den = Uall[:, D:D + 1]
    den = jnp.where(den == 0.0, 1.0, den)
    o = num / den
    mu = jnp.mean(o, axis=1, keepdims=True)
    var = jnp.mean(jnp.square(o - mu), axis=1, keepdims=True)
    o_ref[...] = (o - mu) * lax.rsqrt(var + EPS) * g2_ref[...] + be2_ref[...]


_post = pl.pallas_call(
    _post_body,
    out_shape=jax.ShapeDtypeStruct((N, D), jnp.float32),
)


def kernel(query, edge_index, adj_values, Ww, bw, W1, b1, W2, b2,
           ln1_g, ln1_b, ln2_g, ln2_b):
    ei = edge_index.astype(jnp.int32).reshape(2, NW, NCH, G)
    adjr = adj_values.astype(jnp.float32).reshape(NW, NCH, G)
    va, s1 = _pre(query, Ww, bw.reshape(1, D), W1, b1.reshape(1, D),
                  W2, b2.reshape(1, D), ln1_g.reshape(1, D), ln1_b.reshape(1, D))
    U = _sc(ei, adjr, s1.reshape(N), va)
    return _post(U, ln2_g.reshape(1, D), ln2_b.reshape(1, D))
